# baseline (speedup 1.0000x reference)
import functools

import jax
import jax.numpy as jnp
from jax import lax
from jax.experimental import pallas as pl
from jax.experimental.pallas import tpu as pltpu
from jax.experimental.pallas import tpu_sc as plsc

N = 10000
D = 128
DW = D // 2
E = 320000
NC, NS, L = 2, 16, 16
NW = NC * NS
EPW = E // NW
C = 80
NCH = EPW // C
G = C // L
NBUF = 5
NO = NCH // NBUF
EPS = 1e-6
K1 = D * EPS * EPS


def _normalize_body(z_ref, out_ref, g_ref, f_ref):
    z = z_ref[...]
    n = jnp.sqrt(jnp.sum(z * z, axis=1, keepdims=True))
    ub = (z / n).astype(jnp.bfloat16)
    out_ref[...] = ub
    uf = ub.astype(jnp.float32)
    p = jnp.sum(uf * uf, axis=1, keepdims=True)
    s2 = (2.0 * EPS) * jnp.sum(uf, axis=1, keepdims=True)
    g_ref[...] = jnp.broadcast_to(p + s2, g_ref.shape)
    f_ref[...] = jnp.broadcast_to(p - s2, f_ref.shape)


def _normalize(z):
    blk = N // 10
    zn, g8, f8 = pl.pallas_call(
        _normalize_body,
        out_shape=(
            jax.ShapeDtypeStruct((N, D), jnp.bfloat16),
            jax.ShapeDtypeStruct((N, 8), jnp.float32),
            jax.ShapeDtypeStruct((N, 8), jnp.float32),
        ),
        grid=(10,),
        in_specs=[pl.BlockSpec((blk, D), lambda i: (i, 0))],
        out_specs=(
            pl.BlockSpec((blk, D), lambda i: (i, 0)),
            pl.BlockSpec((blk, 8), lambda i: (i, 0)),
            pl.BlockSpec((blk, 8), lambda i: (i, 0)),
        ),
    )(z)
    table = lax.bitcast_convert_type(zn.reshape(N, DW, 2), jnp.int32)
    return table, g8[:, 0], f8[:, 0]


def _rsqrt_newton(x):
    xi = plsc.bitcast(x, jnp.int32)
    yi = jnp.int32(0x5F3759DF) - (xi >> 1)
    y = plsc.bitcast(yi, jnp.float32)
    for _ in range(3):
        y = y * (1.5 - 0.5 * x * y * y)
    return y


def _edge_body(zn_hbm, g_hbm, f_hbm, src_hbm, dst_hbm, out_hbm,
               si_v, di_v, g_v, f_v, a_bufs, b_bufs, stage_v, o_v, sems):
    wid = lax.axis_index("s") * NC + lax.axis_index("c")
    base = pl.multiple_of(wid * EPW, 8)
    pltpu.sync_copy(src_hbm.at[pl.ds(base, EPW)], si_v)
    pltpu.sync_copy(dst_hbm.at[pl.ds(base, EPW)], di_v)
    pltpu.sync_copy(g_hbm, g_v)
    pltpu.sync_copy(f_hbm, f_v)

    row16 = lax.iota(jnp.int32, 16)
    himask = jnp.full((16,), -65536, jnp.int32)

    def fire(j, b):
        off = pl.multiple_of(j * C, 8)
        pltpu.async_copy(zn_hbm.at[si_v.at[pl.ds(off, C)]], a_bufs[b], sems[b])
        pltpu.async_copy(zn_hbm.at[di_v.at[pl.ds(off, C)]], b_bufs[b], sems[b])

    def drain(b):
        pltpu.make_async_copy(
            zn_hbm.at[si_v.at[pl.ds(0, C)]], a_bufs[b], sems[b]).wait()
        pltpu.make_async_copy(
            zn_hbm.at[di_v.at[pl.ds(0, C)]], b_bufs[b], sems[b]).wait()

    def compute(b, j):
        a_v, b_v = a_bufs[b], b_bufs[b]
        joff = pl.multiple_of(j * C, 8)

        def gbody(gr, carry):
            for u in range(16):
                e = gr * L + u
                acc0 = None
                acc1 = None
                for kk in range(4):
                    va = a_v[e, pl.ds(kk * L, L)]
                    vb = b_v[e, pl.ds(kk * L, L)]
                    a_lo = plsc.bitcast(va << 16, jnp.float32)
                    a_hi = plsc.bitcast(va & himask, jnp.float32)
                    b_lo = plsc.bitcast(vb << 16, jnp.float32)
                    b_hi = plsc.bitcast(vb & himask, jnp.float32)
                    p0 = a_lo * b_lo
                    p1 = a_hi * b_hi
                    acc0 = p0 if acc0 is None else acc0 + p0
                    acc1 = p1 if acc1 is None else acc1 + p1
                stage_v[u, pl.ds(0, L)] = acc0 + acc1

            dots = [None] * 4
            for k in range(16):
                colk = plsc.load_gather(
                    stage_v, [row16, jnp.full((16,), k, jnp.int32)])
                q = k % 4
                dots[q] = colk if dots[q] is None else dots[q] + colk
            dot = (dots[0] + dots[1]) + (dots[2] + dots[3])
            sidx = si_v[pl.ds(joff + gr * L, L)]
            didx = di_v[pl.ds(joff + gr * L, L)]
            gs = plsc.load_gather(g_v, [sidx])
            fd = plsc.load_gather(f_v, [didx])
            x = jnp.maximum(gs + fd - (dot + dot) + K1, 0.0)
            d = x * _rsqrt_newton(x)
            o = 1.0 / (1.0 + jnp.exp(d - 1.0))
            o_v[pl.ds(b * C + gr * L, L)] = o
            return carry

        lax.fori_loop(0, G, gbody, 0)

    for b in range(NBUF):
        fire(b, b)

    def outer(t, carry):
        for b in range(NBUF):
            j = t * NBUF + b
            drain(b)
            compute(b, j)

            @pl.when(j + NBUF < NCH)
            def _():
                fire(j + NBUF, b)

        dst = out_hbm.at[pl.ds(pl.multiple_of(base + t * (NBUF * C), 8),
                               NBUF * C)]
        pltpu.sync_copy(o_v, dst)
        return carry

    lax.fori_loop(0, NO, outer, 0)


_edge_kernel = functools.partial(
    pl.kernel,
    out_type=jax.ShapeDtypeStruct((E,), jnp.float32),
    mesh=plsc.VectorSubcoreMesh(
        core_axis_name="c", subcore_axis_name="s", num_cores=NC, num_subcores=NS
    ),
    scratch_types=[
        pltpu.VMEM((EPW,), jnp.int32),
        pltpu.VMEM((EPW,), jnp.int32),
        pltpu.VMEM((N,), jnp.float32),
        pltpu.VMEM((N,), jnp.float32),
        [pltpu.VMEM((C, DW), jnp.int32) for _ in range(NBUF)],
        [pltpu.VMEM((C, DW), jnp.int32) for _ in range(NBUF)],
        pltpu.VMEM((L, 17), jnp.float32),
        pltpu.VMEM((NBUF * C,), jnp.float32),
        [pltpu.SemaphoreType.DMA for _ in range(NBUF)],
    ],
    compiler_params=pltpu.CompilerParams(
        needs_layout_passes=False, use_tc_tiling_on_sc=False),
)(_edge_body)


@jax.jit
def kernel(z, edge_index):
    table, g, f = _normalize(z)
    return _edge_kernel(table, g, f, edge_index[0], edge_index[1])


# speedup vs baseline: 1.3363x; 1.3363x over previous
import functools

import jax
import jax.numpy as jnp
from jax import lax
from jax.experimental import pallas as pl
from jax.experimental.pallas import tpu as pltpu
from jax.experimental.pallas import tpu_sc as plsc

N = 10000
D = 128
DW = D // 2
E = 320000
NC, NS, L = 2, 16, 16
NW = NC * NS
EPW = E // NW
C = 80
NCH = EPW // C
G = C // L
NBUF = 5
NO = NCH // NBUF
EPS = 1e-6
K1 = D * EPS * EPS


def _normalize_body(z_ref, out_ref, g_ref):
    z = z_ref[...]
    n = jnp.sqrt(jnp.sum(z * z, axis=1, keepdims=True))
    ub = (z / n).astype(jnp.bfloat16)
    out_ref[...] = ub
    uf = ub.astype(jnp.float32)
    p = jnp.sum(uf * uf, axis=1, keepdims=True)
    s2 = (2.0 * EPS) * jnp.sum(uf, axis=1, keepdims=True)
    gb = lax.bitcast_convert_type((p + s2).astype(jnp.bfloat16)
                                  .astype(jnp.float32), jnp.int32)
    fb = lax.bitcast_convert_type((p - s2).astype(jnp.bfloat16)
                                  .astype(jnp.float32), jnp.int32)
    hi = jnp.int32(-65536)
    gf = (gb & hi) | ((fb >> 16) & jnp.int32(0xFFFF))
    g_ref[...] = jnp.broadcast_to(gf, g_ref.shape)


def _normalize(z):
    blk = N // 10
    zn, g8 = pl.pallas_call(
        _normalize_body,
        out_shape=(
            jax.ShapeDtypeStruct((N, D), jnp.bfloat16),
            jax.ShapeDtypeStruct((N, 8), jnp.int32),
        ),
        grid=(10,),
        in_specs=[pl.BlockSpec((blk, D), lambda i: (i, 0))],
        out_specs=(
            pl.BlockSpec((blk, D), lambda i: (i, 0)),
            pl.BlockSpec((blk, 8), lambda i: (i, 0)),
        ),
    )(z)
    table = lax.bitcast_convert_type(zn.reshape(N, DW, 2), jnp.int32)
    return table, g8[:, 0]


def _rsqrt_newton(x):
    xi = plsc.bitcast(x, jnp.int32)
    yi = jnp.int32(0x5F3759DF) - (xi >> 1)
    y = plsc.bitcast(yi, jnp.float32)
    for _ in range(3):
        y = y * (1.5 - 0.5 * x * y * y)
    return y


def _edge_body(zn_hbm, gf_hbm, src_hbm, dst_hbm, out_hbm,
               si_v, di_v, gf_v, a_bufs, b_bufs, sh_table, o_v, sems):
    wid = lax.axis_index("s") * NC + lax.axis_index("c")
    base = pl.multiple_of(wid * EPW, 8)

    @pl.when(lax.axis_index("s") == 0)
    def _():
        pltpu.sync_copy(zn_hbm, sh_table)

    plsc.subcore_barrier()

    pltpu.sync_copy(src_hbm.at[pl.ds(base, EPW)], si_v)
    pltpu.sync_copy(dst_hbm.at[pl.ds(base, EPW)], di_v)
    pltpu.sync_copy(gf_hbm, gf_v)

    row16 = lax.iota(jnp.int32, 16)
    himask = jnp.full((16,), -65536, jnp.int32)

    def fire(j, b):
        off = pl.multiple_of(j * C, 8)
        pltpu.async_copy(
            sh_table.at[si_v.at[pl.ds(off, C)]], a_bufs[b], sems[b])
        pltpu.async_copy(
            sh_table.at[di_v.at[pl.ds(off, C)]], b_bufs[b], sems[b])

    def drain(b):
        pltpu.make_async_copy(
            sh_table.at[si_v.at[pl.ds(0, C)]], a_bufs[b], sems[b]).wait()
        pltpu.make_async_copy(
            sh_table.at[di_v.at[pl.ds(0, C)]], b_bufs[b], sems[b]).wait()

    def compute(b, j):
        a_v, b_v = a_bufs[b], b_bufs[b]
        joff = pl.multiple_of(j * C, 8)

        def gbody(gr, carry):
            def oct_(qq, dotc):
                for u8 in range(8):
                    u = qq * 8 + u8
                    e = gr * L + u
                    acc0 = None
                    acc1 = None
                    for kk in range(4):
                        va = a_v[e, pl.ds(kk * L, L)]
                        vb = b_v[e, pl.ds(kk * L, L)]
                        a_lo = plsc.bitcast(va << 16, jnp.float32)
                        a_hi = plsc.bitcast(va & himask, jnp.float32)
                        b_lo = plsc.bitcast(vb << 16, jnp.float32)
                        b_hi = plsc.bitcast(vb & himask, jnp.float32)
                        p0 = a_lo * b_lo
                        p1 = a_hi * b_hi
                        acc0 = p0 if acc0 is None else acc0 + p0
                        acc1 = p1 if acc1 is None else acc1 + p1
                    dotc = jnp.where(row16 == u, jnp.sum(acc0 + acc1), dotc)
                return dotc

            dot = lax.fori_loop(0, 2, oct_, jnp.zeros((16,), jnp.float32))
            sidx = si_v[pl.ds(joff + gr * L, L)]
            didx = di_v[pl.ds(joff + gr * L, L)]
            gs = plsc.bitcast(plsc.load_gather(gf_v, [sidx]) & himask,
                              jnp.float32)
            fd = plsc.bitcast(plsc.load_gather(gf_v, [didx]) << 16,
                              jnp.float32)
            x = jnp.maximum(gs + fd - (dot + dot) + K1, 0.0)
            d = x * _rsqrt_newton(x)
            o = 1.0 / (1.0 + jnp.exp(d - 1.0))
            o_v[pl.ds(b * C + gr * L, L)] = o
            return carry

        lax.fori_loop(0, G, gbody, 0)

    for b in range(NBUF):
        fire(b, b)

    def outer(t, carry):
        for b in range(NBUF):
            j = t * NBUF + b
            drain(b)
            compute(b, j)

            @pl.when(j + NBUF < NCH)
            def _():
                fire(j + NBUF, b)

        dst = out_hbm.at[pl.ds(pl.multiple_of(base + t * (NBUF * C), 8),
                               NBUF * C)]
        pltpu.sync_copy(o_v, dst)
        return carry

    lax.fori_loop(0, NO, outer, 0)


_edge_kernel = functools.partial(
    pl.kernel,
    out_type=jax.ShapeDtypeStruct((E,), jnp.float32),
    mesh=plsc.VectorSubcoreMesh(
        core_axis_name="c", subcore_axis_name="s", num_cores=NC, num_subcores=NS
    ),
    scratch_types=[
        pltpu.VMEM((EPW,), jnp.int32),
        pltpu.VMEM((EPW,), jnp.int32),
        pltpu.VMEM((N,), jnp.int32),
        [pltpu.VMEM((C, DW), jnp.int32) for _ in range(NBUF)],
        [pltpu.VMEM((C, DW), jnp.int32) for _ in range(NBUF)],
        pltpu.VMEM_SHARED((N, DW), jnp.int32),
        pltpu.VMEM((NBUF * C,), jnp.float32),
        [pltpu.SemaphoreType.DMA for _ in range(NBUF)],
    ],
    compiler_params=pltpu.CompilerParams(
        needs_layout_passes=False, use_tc_tiling_on_sc=False),
)(_edge_body)


@jax.jit
def kernel(z, edge_index):
    table, gf = _normalize(z)
    return _edge_kernel(table, gf, edge_index[0], edge_index[1])


# speedup vs baseline: 1.4673x; 1.0980x over previous
import functools

import jax
import jax.numpy as jnp
from jax import lax
from jax.experimental import pallas as pl
from jax.experimental.pallas import tpu as pltpu
from jax.experimental.pallas import tpu_sc as plsc

N = 10000
D = 128
E = 320000
NC, NS, L = 2, 16, 16
NW = NC * NS
EPW = E // NW
C = 80
NCH = EPW // C
G = C // L
NBUF = 5
NO = NCH // NBUF
EPS = 1e-6


def _normalize_body(z_ref, out_ref):
    z = z_ref[...]
    n = jnp.sqrt(jnp.sum(z * z, axis=1, keepdims=True))
    out_ref[...] = z / n


def _normalize(z):
    blk = N // 10
    return pl.pallas_call(
        _normalize_body,
        out_shape=jax.ShapeDtypeStruct((N, D), jnp.float32),
        grid=(10,),
        in_specs=[pl.BlockSpec((blk, D), lambda i: (i, 0))],
        out_specs=pl.BlockSpec((blk, D), lambda i: (i, 0)),
    )(z)


def _rsqrt_newton(x):
    xi = plsc.bitcast(x, jnp.int32)
    yi = jnp.int32(0x5F3759DF) - (xi >> 1)
    y = plsc.bitcast(yi, jnp.float32)
    for _ in range(3):
        y = y * (1.5 - 0.5 * x * y * y)
    return y


def _edge_body(zn_hbm, src_hbm, dst_hbm, out_hbm, si_v, di_v, a_bufs, b_bufs,
               o_v, sems, o_sem):
    wid = lax.axis_index("s") * NC + lax.axis_index("c")
    base = pl.multiple_of(wid * EPW, 8)
    pltpu.sync_copy(src_hbm.at[pl.ds(base, EPW)], si_v)
    pltpu.sync_copy(dst_hbm.at[pl.ds(base, EPW)], di_v)

    row16 = lax.iota(jnp.int32, 16)

    def fire(j, b):
        off = pl.multiple_of(j * C, 8)
        pltpu.async_copy(zn_hbm.at[si_v.at[pl.ds(off, C)]], a_bufs[b], sems[b])
        pltpu.async_copy(zn_hbm.at[di_v.at[pl.ds(off, C)]], b_bufs[b], sems[b])

    def drain(b):
        pltpu.make_async_copy(
            zn_hbm.at[si_v.at[pl.ds(0, C)]], a_bufs[b], sems[b]).wait()
        pltpu.make_async_copy(
            zn_hbm.at[di_v.at[pl.ds(0, C)]], b_bufs[b], sems[b]).wait()

    def o_dst(t):
        return out_hbm.at[pl.ds(pl.multiple_of(base + t * (NBUF * C), 8),
                                NBUF * C)]

    def compute(b):
        a_v, b_v = a_bufs[b], b_bufs[b]

        def gbody(g, carry):
            def quad(qq, x):
                for u4 in range(4):
                    u = qq * 4 + u4
                    e = g * L + u
                    acc = None
                    for kk in range(8):
                        va = a_v[e, pl.ds(kk * L, L)]
                        vb = b_v[e, pl.ds(kk * L, L)]
                        t = va - vb + EPS
                        p = t * t
                        acc = p if acc is None else acc + p
                    x = jnp.where(row16 == u, jnp.sum(acc), x)
                return x

            x = lax.fori_loop(0, 4, quad, jnp.zeros((16,), jnp.float32))
            d = x * _rsqrt_newton(x)
            o = 1.0 / (1.0 + jnp.exp(d - 1.0))
            o_v[pl.ds(b * C + g * L, L)] = o
            return carry

        lax.fori_loop(0, G, gbody, 0)

    for b in range(NBUF):
        fire(b, b)

    def outer(t, carry):
        @pl.when(t > 0)
        def _():
            pltpu.make_async_copy(o_v, o_dst(0), o_sem).wait()

        for b in range(NBUF):
            j = t * NBUF + b
            drain(b)
            compute(b)

            @pl.when(j + NBUF < NCH)
            def _():
                fire(j + NBUF, b)

        pltpu.async_copy(o_v, o_dst(t), o_sem)
        return carry

    lax.fori_loop(0, NO, outer, 0)
    pltpu.make_async_copy(o_v, o_dst(0), o_sem).wait()


_edge_kernel = functools.partial(
    pl.kernel,
    out_type=jax.ShapeDtypeStruct((E,), jnp.float32),
    mesh=plsc.VectorSubcoreMesh(
        core_axis_name="c", subcore_axis_name="s", num_cores=NC, num_subcores=NS
    ),
    scratch_types=[
        pltpu.VMEM((EPW,), jnp.int32),
        pltpu.VMEM((EPW,), jnp.int32),
        [pltpu.VMEM((C, D), jnp.float32) for _ in range(NBUF)],
        [pltpu.VMEM((C, D), jnp.float32) for _ in range(NBUF)],
        pltpu.VMEM((NBUF * C,), jnp.float32),
        [pltpu.SemaphoreType.DMA for _ in range(NBUF)],
        pltpu.SemaphoreType.DMA,
    ],
    compiler_params=pltpu.CompilerParams(needs_layout_passes=False),
)(_edge_body)


@jax.jit
def kernel(z, edge_index):
    zn = _normalize(z)
    return _edge_kernel(zn, edge_index[0], edge_index[1])
